# baseline (speedup 1.0000x reference)
import functools

import jax
import jax.numpy as jnp
from jax import lax
from jax.experimental import pallas as pl
from jax.experimental.pallas import tpu as pltpu
from jax.experimental.pallas import tpu_sc as plsc

_K_HIGH = 16
_K_LOW = 2


def _topk_body(sig_ref, x_ref, idx_ref, deg_ref, sim_ref, *, bt: int, t: int,
               zero_idx: int):
    b = pl.program_id(0)
    i = pl.program_id(1)
    xk = x_ref[0]
    q = x_ref[0, pl.ds(i * bt, bt), :]

    kn = xk / jnp.clip(jnp.sqrt(jnp.sum(xk * xk, axis=1, keepdims=True)),
                       1e-12, None)
    qn = q / jnp.clip(jnp.sqrt(jnp.sum(q * q, axis=1, keepdims=True)),
                      1e-12, None)

    sim = jax.lax.dot_general(qn, kn, (((1,), (1,)), ((), ())),
                              preferred_element_type=jnp.float32)

    iota_s = jax.lax.broadcasted_iota(jnp.int32, (bt, t), 1)
    t_glob = i * bt + jax.lax.broadcasted_iota(jnp.int32, (bt, t), 0)
    sim_ref[...] = jnp.where(iota_s < t_glob, sim, jnp.float32(-1e9))

    sigma = sig_ref[0, 0]
    surp = jnp.tanh(sigma * jnp.mean(jnp.abs(q), axis=1, keepdims=True))
    kt = jnp.clip(jnp.round(_K_LOW + (_K_HIGH - _K_LOW) * surp),
                  0.0, float(min(_K_HIGH, t - 1)))

    vals, idxs = [], []
    for j in range(_K_HIGH):
        s = sim_ref[...]
        cur = jnp.max(s, axis=1, keepdims=True)
        idx = jnp.argmax(s, axis=1).reshape(bt, 1)
        sim_ref[...] = jnp.where(iota_s == idx, jnp.float32(-2e9), s)
        vals.append(cur)
        idxs.append(idx)
    v16 = jnp.concatenate(vals, axis=1)
    i16 = jnp.concatenate(idxs, axis=1)

    jj = jax.lax.broadcasted_iota(jnp.int32, (bt, _K_HIGH), 1)
    kti = kt.astype(jnp.int32)
    sel = jnp.logical_and(jj < kti, v16 > -1e8)
    gidx = jnp.where(sel, i16 + b * t, jnp.int32(zero_idx))
    idx_ref[0] = gidx
    deg_ref[0] = jnp.maximum(jnp.sum(sel.astype(jnp.float32), axis=1,
                                     keepdims=True), 1.0)


def _sc_gather_sum(xtab, idxf, *, n_tok: int, d: int):
    info = plsc.get_sparse_core_info()
    nw = info.num_cores * info.num_subcores
    tok_w = n_tok // nw
    mesh = plsc.VectorSubcoreMesh(core_axis_name="c", subcore_axis_name="s")

    @functools.partial(
        pl.kernel, mesh=mesh,
        out_type=jax.ShapeDtypeStruct((n_tok, d), jnp.float32),
        scratch_types=[
            pltpu.VMEM((tok_w, _K_HIGH), jnp.int32),
            pltpu.VMEM((_K_HIGH, d), jnp.float32),
            pltpu.VMEM((_K_HIGH, d), jnp.float32),
            pltpu.VMEM((d,), jnp.float32),
            pltpu.SemaphoreType.DMA,
            pltpu.SemaphoreType.DMA,
        ],
    )
    def k(xtab_hbm, idx_hbm, out_hbm, idx_v, buf0, buf1, msg_v, sem0, sem1):
        wid = lax.axis_index("s") * info.num_cores + lax.axis_index("c")
        base = wid * tok_w
        pltpu.sync_copy(idx_hbm.at[pl.ds(base, tok_w)], idx_v)

        bufs = (buf0, buf1)
        sems = (sem0, sem1)
        pltpu.make_async_copy(xtab_hbm.at[idx_v[0, :]], buf0, sem0).start()

        def step(tk, _):
            for par in range(2):
                cur_buf, cur_sem = bufs[par], sems[par]
                nxt_buf, nxt_sem = bufs[1 - par], sems[1 - par]
                tok = tk + par
                pltpu.make_async_copy(
                    xtab_hbm.at[idx_v[tok, :]], cur_buf, cur_sem).wait()
                @pl.when(tok + 1 < tok_w)
                def _():
                    pltpu.make_async_copy(
                        xtab_hbm.at[idx_v[tok + 1, :]], nxt_buf,
                        nxt_sem).start()

                def col(cc, _):
                    acc = cur_buf[0, pl.ds(cc * 16, 16)]
                    for r in range(1, _K_HIGH):
                        acc = acc + cur_buf[r, pl.ds(cc * 16, 16)]
                    msg_v[pl.ds(cc * 16, 16)] = acc
                    return 0
                lax.fori_loop(0, d // 16, col, 0, unroll=4)
                pltpu.sync_copy(msg_v, out_hbm.at[base + tok])
            return 0
        lax.fori_loop(0, tok_w // 2, lambda g, c: step(2 * g, c), 0)

    return k(xtab, idxf)


def _blend_body(mix_ref, scl_ref, x_ref, m_ref, d_ref, gain_ref, bias_ref,
                out_ref, *, bt: int, t: int):
    x = x_ref[0]
    msum = m_ref[0]
    deg = d_ref[0]

    mix = mix_ref[0, 0]
    scale = scl_ref[0, 0]
    blended = mix * x + (1.0 - mix) * (msum / deg)
    y = blended * gain_ref[0] + bias_ref[0]
    gelu = 0.5 * y * (1.0 + jax.lax.erf(y * jnp.float32(0.7071067811865476)))
    out_ref[0] = gelu * scale


def _stage_a(x, sigma, *, bt: int, interpret: bool = False):
    b, t, d = x.shape
    return pl.pallas_call(
        functools.partial(_topk_body, bt=bt, t=t, zero_idx=b * t),
        grid=(b, t // bt),
        in_specs=[
            pl.BlockSpec((1, 1), lambda bb, ii: (0, 0),
                         memory_space=pltpu.SMEM),
            pl.BlockSpec((1, t, d), lambda bb, ii: (bb, 0, 0)),
        ],
        out_specs=[
            pl.BlockSpec((1, bt, _K_HIGH), lambda bb, ii: (bb, ii, 0)),
            pl.BlockSpec((1, bt, 1), lambda bb, ii: (bb, ii, 0)),
        ],
        out_shape=[
            jax.ShapeDtypeStruct((b, t, _K_HIGH), jnp.int32),
            jax.ShapeDtypeStruct((b, t, 1), jnp.float32),
        ],
        scratch_shapes=[pltpu.VMEM((bt, t), jnp.float32)],
        interpret=interpret,
    )(sigma, x)


def _stage_c(x, msum, deg, mix, scale, gain, bias, *, bt: int,
             interpret: bool = False):
    b, t, d = x.shape
    return pl.pallas_call(
        functools.partial(_blend_body, bt=bt, t=t),
        grid=(b, t // bt),
        in_specs=[
            pl.BlockSpec((1, 1), lambda bb, ii: (0, 0),
                         memory_space=pltpu.SMEM),
            pl.BlockSpec((1, 1), lambda bb, ii: (0, 0),
                         memory_space=pltpu.SMEM),
            pl.BlockSpec((1, bt, d), lambda bb, ii: (bb, ii, 0)),
            pl.BlockSpec((1, bt, d), lambda bb, ii: (bb, ii, 0)),
            pl.BlockSpec((1, bt, 1), lambda bb, ii: (bb, ii, 0)),
            pl.BlockSpec((1, d), lambda bb, ii: (0, 0)),
            pl.BlockSpec((1, d), lambda bb, ii: (0, 0)),
        ],
        out_specs=pl.BlockSpec((1, bt, d), lambda bb, ii: (bb, ii, 0)),
        out_shape=jax.ShapeDtypeStruct((b, t, d), jnp.float32),
        interpret=interpret,
    )(mix, scale, x, msum, deg, gain, bias)


@jax.jit
def kernel(x, gain, bias, log_sigma_raw, log_mix, log_scale):
    b, t, d = x.shape
    bt = 256

    sigma = (jax.nn.softplus(log_sigma_raw) + 0.01).reshape(1, 1)
    mix = jax.nn.sigmoid(log_mix).reshape(1, 1)
    scale = (jax.nn.softplus(log_scale) + 0.01).reshape(1, 1)
    sigma = sigma.astype(jnp.float32)

    idx, deg = _stage_a(x, sigma, bt=bt)
    xtab = jnp.concatenate(
        [x.reshape(b * t, d), jnp.zeros((8, d), jnp.float32)], axis=0)
    msum = _sc_gather_sum(xtab, idx.reshape(b * t, _K_HIGH),
                          n_tok=b * t, d=d)
    return _stage_c(x, msum.reshape(b, t, d), deg,
                    mix.astype(jnp.float32), scale.astype(jnp.float32),
                    gain.reshape(1, d), bias.reshape(1, d), bt=bt)
